# Initial kernel scaffold; baseline (speedup 1.0000x reference)
import jax
import jax.numpy as jnp
from jax.experimental import pallas as pl


def kernel(pillar_features, coords):
    raise NotImplementedError("write your pallas kernel here")


# speedup vs baseline: 1.3435x; 1.3435x over previous
import functools

import jax
import jax.numpy as jnp
from jax import lax
from jax.experimental import pallas as pl
from jax.experimental.pallas import tpu as pltpu, tpu_sc as plsc

F = 64
NX, NY = 432, 496
S = NX * NY
B = 4
P = 80000
NC, NS = 2, 16
NW = NC * NS
CHUNK = 256
CPB = S // CHUNK
NCHUNK = B * CPB
TRIP_MAX = -(-NCHUNK // NW)
R = TRIP_MAX * CHUNK
CP = 8000
NCP = P // CP
G = 128
PIDCAP = CHUNK
STAGE = 4


def _body(bcol, ycol, xcol, feats, out,
          slotmap, bbuf, ybuf, xbuf, blk, rows, pidl, offl, sem):
    wid = lax.axis_index("s") * NC + lax.axis_index("c")
    ntrip = jnp.where(wid < NCHUNK - (TRIP_MAX - 1) * NW, TRIP_MAX, TRIP_MAX - 1)
    sent = P + wid
    iota = lax.iota(jnp.int32, 16)
    zf = jnp.zeros((16,), jnp.float32)
    sentv = jnp.full((16,), sent, dtype=jnp.int32)

    def init_map(i, _):
        slotmap[pl.ds(pl.multiple_of(i * 16, 16), 16)] = sentv
        return 0
    lax.fori_loop(0, R // 16, init_map, 0)

    def init_blk(f, _):
        def inner(i, _):
            blk[f, pl.ds(pl.multiple_of(i * 16, 16), 16)] = zf
            return 0
        lax.fori_loop(0, CHUNK // 16, inner, 0)
        return 0
    lax.fori_loop(0, F, init_blk, 0)

    if STAGE < 2:
        return
    def p1_chunk(c, _):
        base = pl.multiple_of(c * CP, CP)
        pltpu.sync_copy(bcol.at[pl.ds(base, CP)], bbuf)
        pltpu.sync_copy(ycol.at[pl.ds(base, CP)], ybuf)
        pltpu.sync_copy(xcol.at[pl.ds(base, CP)], xbuf)

        def p1_vec(i, _):
            o = pl.ds(pl.multiple_of(i * 16, 16), 16)
            g = bbuf[o] * S + ybuf[o] * NX + xbuf[o]
            ch = lax.shift_right_logical(g, 8)
            valid = (ch & (NW - 1)) == wid
            loc = lax.shift_left(lax.shift_right_logical(ch, 5), 8) | (g & (CHUNK - 1))
            locc = jnp.where(valid, loc, 0)
            pv = base + i * 16 + iota
            plsc.store_scatter(slotmap, [locc], pv, mask=valid)
            return 0
        lax.fori_loop(0, CP // 16, p1_vec, 0)
        return 0
    lax.fori_loop(0, NCP, p1_chunk, 0)

    if STAGE < 3:
        return
    def p2_chunk(k, _):
        mbase = pl.multiple_of(k * CHUNK, CHUNK)
        c = wid + k * NW
        batch = c // CPB
        sbase = (c - batch * CPB) * CHUNK

        def pad(i, _):
            o = pl.ds(pl.multiple_of(i * 16, 16), 16)
            pidl[o] = sentv
            return 0
        lax.fori_loop(0, PIDCAP // 16, pad, 0)

        cntv = jnp.zeros((16,), jnp.int32)
        for i in range(CHUNK // 16):
            o = pl.ds(pl.multiple_of(mbase + i * 16, 16), 16)
            m = slotmap[o]
            valid = m < P
            cums = plsc.cumsum(valid.astype(jnp.int32))
            off = cntv + cums - 1
            plsc.store_scatter(pidl, [off], m, mask=valid)
            plsc.store_scatter(offl, [off], i * 16 + iota, mask=valid)
            cntv = cntv + plsc.all_reduce_population_count(valid)
        n = jnp.max(cntv)
        nseg = (n + (G - 1)) // G

        if STAGE < 4:
            pltpu.sync_copy(blk, out.at[batch, :, pl.ds(sbase, CHUNK)])
            return 0
        def seg_body(sg, _):
            sgo = pl.multiple_of(sg * G, G)
            pltpu.async_copy(feats.at[pidl.at[pl.ds(sgo, G)]], rows, sem).wait()
            ngrp = (jnp.minimum(n - sgo, G) + 15) // 16
            def grp(gi, _):
                q0 = pl.multiple_of(sgo + gi * 16, 16)
                maskv = (q0 + iota) < n
                offv = offl[pl.ds(q0, 16)]
                rloc = pl.multiple_of(gi * 16, 16) + iota
                for f in range(F):
                    fv = jnp.full((16,), f, dtype=jnp.int32)
                    val = plsc.load_gather(rows, [rloc, fv])
                    plsc.store_scatter(blk, [fv, offv], val, mask=maskv)
                return 0
            lax.fori_loop(0, ngrp, grp, 0)
            return 0
        lax.fori_loop(0, nseg, seg_body, 0)

        pltpu.sync_copy(blk, out.at[batch, :, pl.ds(sbase, CHUNK)])

        def scrub(gi, _):
            q0 = pl.multiple_of(gi * 16, 16)
            maskv = (q0 + iota) < n
            offv = offl[pl.ds(q0, 16)]
            for f in range(F):
                fv = jnp.full((16,), f, dtype=jnp.int32)
                plsc.store_scatter(blk, [fv, offv], zf, mask=maskv)
            return 0
        lax.fori_loop(0, (n + 15) // 16, scrub, 0)
        return 0
    lax.fori_loop(0, ntrip, p2_chunk, 0)


@jax.jit
def kernel(pillar_features, coords):
    feats = jnp.zeros((P + NW, 128), pillar_features.dtype)
    feats = feats.at[:P, :F].set(pillar_features)
    bcol = coords[:, 0]
    ycol = coords[:, 2]
    xcol = coords[:, 3]
    mesh = plsc.VectorSubcoreMesh(core_axis_name="c", subcore_axis_name="s")
    run = functools.partial(
        pl.kernel,
        mesh=mesh,
        compiler_params=pltpu.CompilerParams(needs_layout_passes=False),
        out_type=jax.ShapeDtypeStruct((B, F, S), jnp.float32),
        scratch_types=[
            pltpu.VMEM((R,), jnp.int32),
            pltpu.VMEM((CP,), jnp.int32),
            pltpu.VMEM((CP,), jnp.int32),
            pltpu.VMEM((CP,), jnp.int32),
            pltpu.VMEM((F, CHUNK), jnp.float32),
            pltpu.VMEM((G, 128), jnp.float32),
            pltpu.VMEM((PIDCAP,), jnp.int32),
            pltpu.VMEM((PIDCAP,), jnp.int32),
            pltpu.SemaphoreType.DMA,
        ],
    )(_body)
    out = run(bcol, ycol, xcol, feats)
    return out.reshape(B, F, NY, NX)


# speedup vs baseline: 1.3602x; 1.0124x over previous
import functools

import jax
import jax.numpy as jnp
from jax import lax
from jax.experimental import pallas as pl
from jax.experimental.pallas import tpu as pltpu, tpu_sc as plsc

F = 64
NX, NY = 432, 496
S = NX * NY
B = 4
P = 80000
NC, NS = 2, 16
NW = NC * NS
CHUNK = 256
CPB = S // CHUNK
NCHUNK = B * CPB
TRIP_MAX = -(-NCHUNK // NW)
R = TRIP_MAX * CHUNK
CP = 8000
NCP = P // CP
G = 128
PIDCAP = CHUNK
STAGE = 4


def _body(bcol, ycol, xcol, feats, out,
          slotmap, bbuf, ybuf, xbuf, blk, rows, pidl, offl, sem):
    wid = lax.axis_index("s") * NC + lax.axis_index("c")
    ntrip = jnp.where(wid < NCHUNK - (TRIP_MAX - 1) * NW, TRIP_MAX, TRIP_MAX - 1)
    sent = P + wid
    iota = lax.iota(jnp.int32, 16)
    zf = jnp.zeros((16,), jnp.float32)
    sentv = jnp.full((16,), sent, dtype=jnp.int32)

    def init_map(i, _):
        slotmap[pl.ds(pl.multiple_of(i * 16, 16), 16)] = sentv
        return 0
    lax.fori_loop(0, R // 16, init_map, 0)

    def init_blk(f, _):
        def inner(i, _):
            blk[f, pl.ds(pl.multiple_of(i * 16, 16), 16)] = zf
            return 0
        lax.fori_loop(0, CHUNK // 16, inner, 0)
        return 0
    lax.fori_loop(0, F, init_blk, 0)

    if STAGE < 2:
        return
    def p1_chunk(c, _):
        base = pl.multiple_of(c * CP, CP)
        pltpu.sync_copy(bcol.at[pl.ds(base, CP)], bbuf)
        pltpu.sync_copy(ycol.at[pl.ds(base, CP)], ybuf)
        pltpu.sync_copy(xcol.at[pl.ds(base, CP)], xbuf)

        def p1_vec(i, _):
            o = pl.ds(pl.multiple_of(i * 16, 16), 16)
            g = bbuf[o] * S + ybuf[o] * NX + xbuf[o]
            ch = lax.shift_right_logical(g, 8)
            valid = (ch & (NW - 1)) == wid
            loc = lax.shift_left(lax.shift_right_logical(ch, 5), 8) | (g & (CHUNK - 1))
            locc = jnp.where(valid, loc, 0)
            pv = base + i * 16 + iota
            plsc.store_scatter(slotmap, [locc], pv, mask=valid)
            return 0
        lax.fori_loop(0, CP // 16, p1_vec, 0)
        return 0
    lax.fori_loop(0, NCP, p1_chunk, 0)

    if STAGE < 3:
        return
    def p2_chunk(k, _):
        mbase = pl.multiple_of(k * CHUNK, CHUNK)
        c = wid + k * NW
        batch = c // CPB
        sbase = (c - batch * CPB) * CHUNK

        def pad(i, _):
            o = pl.ds(pl.multiple_of(i * 16, 16), 16)
            pidl[o] = jnp.full((16,), wid, dtype=jnp.int32)
            return 0
        lax.fori_loop(0, PIDCAP // 16, pad, 0)

        cntv = jnp.zeros((16,), jnp.int32)
        for i in range(CHUNK // 16):
            o = pl.ds(pl.multiple_of(mbase + i * 16, 16), 16)
            m = slotmap[o]
            valid = m < P
            cums = plsc.cumsum(valid.astype(jnp.int32))
            off = cntv + cums - 1
            plsc.store_scatter(pidl, [off], lax.shift_right_logical(m, 1),
                               mask=valid)
            plsc.store_scatter(offl, [off],
                               lax.shift_left(m & 1, 8) | (i * 16 + iota),
                               mask=valid)
            cntv = cntv + plsc.all_reduce_population_count(valid)
        n = jnp.max(cntv)
        nseg = (n + (G - 1)) // G

        if STAGE < 4:
            pltpu.sync_copy(blk, out.at[batch, :, pl.ds(sbase, CHUNK)])
            return 0
        def seg_body(sg, _):
            sgo = pl.multiple_of(sg * G, G)
            pltpu.async_copy(feats.at[pidl.at[pl.ds(sgo, G)]], rows, sem).wait()
            ngrp = (jnp.minimum(n - sgo, G) + 15) // 16
            def grp(gi, _):
                q0 = pl.multiple_of(sgo + gi * 16, 16)
                maskv = (q0 + iota) < n
                offraw = offl[pl.ds(q0, 16)]
                halfcol = lax.shift_left(
                    lax.shift_right_logical(offraw, 8), 6)
                offv = offraw & (CHUNK - 1)
                rloc = pl.multiple_of(gi * 16, 16) + iota
                for f in range(F):
                    fv = jnp.full((16,), f, dtype=jnp.int32)
                    val = plsc.load_gather(rows, [rloc, halfcol + f])
                    plsc.store_scatter(blk, [fv, offv], val, mask=maskv)
                return 0
            lax.fori_loop(0, ngrp, grp, 0)
            return 0
        lax.fori_loop(0, nseg, seg_body, 0)

        pltpu.sync_copy(blk, out.at[batch, :, pl.ds(sbase, CHUNK)])

        def scrub(gi, _):
            q0 = pl.multiple_of(gi * 16, 16)
            maskv = (q0 + iota) < n
            offv = offl[pl.ds(q0, 16)] & (CHUNK - 1)
            for f in range(F):
                fv = jnp.full((16,), f, dtype=jnp.int32)
                plsc.store_scatter(blk, [fv, offv], zf, mask=maskv)
            return 0
        lax.fori_loop(0, (n + 15) // 16, scrub, 0)
        return 0
    lax.fori_loop(0, ntrip, p2_chunk, 0)


@jax.jit
def kernel(pillar_features, coords):
    feats = pillar_features.reshape(P // 2, 2 * F)
    bcol = coords[:, 0]
    ycol = coords[:, 2]
    xcol = coords[:, 3]
    mesh = plsc.VectorSubcoreMesh(core_axis_name="c", subcore_axis_name="s")
    run = functools.partial(
        pl.kernel,
        mesh=mesh,
        compiler_params=pltpu.CompilerParams(needs_layout_passes=False),
        out_type=jax.ShapeDtypeStruct((B, F, S), jnp.float32),
        scratch_types=[
            pltpu.VMEM((R,), jnp.int32),
            pltpu.VMEM((CP,), jnp.int32),
            pltpu.VMEM((CP,), jnp.int32),
            pltpu.VMEM((CP,), jnp.int32),
            pltpu.VMEM((F, CHUNK), jnp.float32),
            pltpu.VMEM((G, 128), jnp.float32),
            pltpu.VMEM((PIDCAP,), jnp.int32),
            pltpu.VMEM((PIDCAP,), jnp.int32),
            pltpu.SemaphoreType.DMA,
        ],
    )(_body)
    out = run(bcol, ycol, xcol, feats)
    return out.reshape(B, F, NY, NX)


# speedup vs baseline: 4.0910x; 3.0077x over previous
import functools

import jax
import jax.numpy as jnp
from jax import lax
from jax.experimental import pallas as pl
from jax.experimental.pallas import tpu as pltpu, tpu_sc as plsc

F = 64
NX, NY = 432, 496
S = NX * NY
B = 4
P = 80000
NC, NS = 2, 16
NW = NC * NS
CHUNK = 1024
NG = (B * S) // CHUNK
TRIP_MAX = -(-NG // NW)
R = TRIP_MAX * CHUNK
CP = 4000
NCP = P // CP
G = 128
PIDCAP = CHUNK
STAGE = 4


def _body(bcol, ycol, xcol, feats, out,
          slotmap, bbuf, ybuf, xbuf, blk, rows, pidl, offl, sem):
    wid = lax.axis_index("s") * NC + lax.axis_index("c")
    ntrip = jnp.where(wid < NG - (TRIP_MAX - 1) * NW, TRIP_MAX, TRIP_MAX - 1)
    sent = P + wid
    iota = lax.iota(jnp.int32, 16)
    zf = jnp.zeros((16,), jnp.float32)
    sentv = jnp.full((16,), sent, dtype=jnp.int32)

    def init_map(i, _):
        slotmap[pl.ds(pl.multiple_of(i * 16, 16), 16)] = sentv
        return 0
    lax.fori_loop(0, R // 16, init_map, 0)

    def init_blk(f, _):
        def inner(i, _):
            blk[f, pl.ds(pl.multiple_of(i * 16, 16), 16)] = zf
            return 0
        lax.fori_loop(0, CHUNK // 16, inner, 0)
        return 0
    lax.fori_loop(0, F, init_blk, 0)

    if STAGE < 2:
        return
    def p1_chunk(c, _):
        base = pl.multiple_of(c * CP, CP)
        pltpu.sync_copy(bcol.at[pl.ds(base, CP)], bbuf)
        pltpu.sync_copy(ycol.at[pl.ds(base, CP)], ybuf)
        pltpu.sync_copy(xcol.at[pl.ds(base, CP)], xbuf)

        def p1_vec(i, _):
            o = pl.ds(pl.multiple_of(i * 16, 16), 16)
            g = bbuf[o] * S + xbuf[o] * NY + ybuf[o]
            gr = lax.shift_right_logical(g, 10)
            valid = (gr & (NW - 1)) == wid
            loc = (lax.shift_left(lax.shift_right_logical(gr, 5), 10)
                   | (g & (CHUNK - 1)))
            locc = jnp.where(valid, loc, 0)
            pv = base + i * 16 + iota
            plsc.store_scatter(slotmap, [locc], pv, mask=valid)
            return 0
        lax.fori_loop(0, CP // 16, p1_vec, 0)
        return 0
    lax.fori_loop(0, NCP, p1_chunk, 0)

    if STAGE < 3:
        return
    def p2_group(k, _):
        mbase = pl.multiple_of(k * CHUNK, CHUNK)
        cid = wid + k * NW
        gs0 = cid * CHUNK
        b0 = gs0 // S
        r0 = gs0 - b0 * S
        rem = S - r0

        def pad(i, _):
            o = pl.ds(pl.multiple_of(i * 16, 16), 16)
            pidl[o] = jnp.full((16,), wid, dtype=jnp.int32)
            return 0
        lax.fori_loop(0, PIDCAP // 16, pad, 0)

        cntv = jnp.zeros((16,), jnp.int32)
        for i in range(CHUNK // 16):
            o = pl.ds(pl.multiple_of(mbase + i * 16, 16), 16)
            m = slotmap[o]
            valid = m < P
            cums = plsc.cumsum(valid.astype(jnp.int32))
            off = cntv + cums - 1
            plsc.store_scatter(pidl, [off], lax.shift_right_logical(m, 1),
                               mask=valid)
            plsc.store_scatter(offl, [off],
                               lax.shift_left(m & 1, 10) | (i * 16 + iota),
                               mask=valid)
            cntv = cntv + plsc.all_reduce_population_count(valid)
        n = jnp.max(cntv)
        nseg = (n + (G - 1)) // G

        def seg_body(sg, _):
            sgo = pl.multiple_of(sg * G, G)
            pltpu.async_copy(feats.at[pidl.at[pl.ds(sgo, G)]], rows, sem).wait()
            ngrp = (jnp.minimum(n - sgo, G) + 15) // 16
            def grp(gi, _):
                q0 = pl.multiple_of(sgo + gi * 16, 16)
                maskv = (q0 + iota) < n
                offraw = offl[pl.ds(q0, 16)]
                halfcol = lax.shift_left(
                    lax.shift_right_logical(offraw, 10), 6)
                offv = offraw & (CHUNK - 1)
                rloc = pl.multiple_of(gi * 16, 16) + iota
                for f in range(F):
                    fv = jnp.full((16,), f, dtype=jnp.int32)
                    val = plsc.load_gather(rows, [rloc, halfcol + f])
                    plsc.store_scatter(blk, [fv, offv], val, mask=maskv)
                return 0
            lax.fori_loop(0, ngrp, grp, 0)
            return 0
        lax.fori_loop(0, nseg, seg_body, 0)

        @pl.when(rem >= CHUNK)
        def _():
            pltpu.sync_copy(blk, out.at[b0, :, pl.ds(r0, CHUNK)])

        @pl.when(rem == 256)
        def _():
            pltpu.sync_copy(blk.at[:, pl.ds(0, 256)],
                            out.at[b0, :, pl.ds(r0, 256)])
            pltpu.sync_copy(blk.at[:, pl.ds(256, 768)],
                            out.at[b0 + 1, :, pl.ds(0, 768)])

        @pl.when(rem == 512)
        def _():
            pltpu.sync_copy(blk.at[:, pl.ds(0, 512)],
                            out.at[b0, :, pl.ds(r0, 512)])
            pltpu.sync_copy(blk.at[:, pl.ds(512, 512)],
                            out.at[b0 + 1, :, pl.ds(0, 512)])

        @pl.when(rem == 768)
        def _():
            pltpu.sync_copy(blk.at[:, pl.ds(0, 768)],
                            out.at[b0, :, pl.ds(r0, 768)])
            pltpu.sync_copy(blk.at[:, pl.ds(768, 256)],
                            out.at[b0 + 1, :, pl.ds(0, 256)])

        def scrub(gi, _):
            q0 = pl.multiple_of(gi * 16, 16)
            maskv = (q0 + iota) < n
            offv = offl[pl.ds(q0, 16)] & (CHUNK - 1)
            for f in range(F):
                fv = jnp.full((16,), f, dtype=jnp.int32)
                plsc.store_scatter(blk, [fv, offv], zf, mask=maskv)
            return 0
        lax.fori_loop(0, (n + 15) // 16, scrub, 0)
        return 0
    lax.fori_loop(0, ntrip, p2_group, 0)


@jax.jit
def kernel(pillar_features, coords):
    feats = pillar_features.reshape(P // 2, 2 * F)
    bcol = coords[:, 0]
    ycol = coords[:, 2]
    xcol = coords[:, 3]
    mesh = plsc.VectorSubcoreMesh(core_axis_name="c", subcore_axis_name="s")
    run = functools.partial(
        pl.kernel,
        mesh=mesh,
        compiler_params=pltpu.CompilerParams(needs_layout_passes=False),
        out_type=jax.ShapeDtypeStruct((B, F, S), jnp.float32),
        scratch_types=[
            pltpu.VMEM((R,), jnp.int32),
            pltpu.VMEM((CP,), jnp.int32),
            pltpu.VMEM((CP,), jnp.int32),
            pltpu.VMEM((CP,), jnp.int32),
            pltpu.VMEM((F, CHUNK), jnp.float32),
            pltpu.VMEM((G, 2 * F), jnp.float32),
            pltpu.VMEM((PIDCAP,), jnp.int32),
            pltpu.VMEM((PIDCAP,), jnp.int32),
            pltpu.SemaphoreType.DMA,
        ],
    )(_body)
    out = run(bcol, ycol, xcol, feats)
    return jnp.swapaxes(out.reshape(B, F, NX, NY), 2, 3)
